# baseline (speedup 1.0000x reference)
import functools

import jax
import jax.numpy as jnp
from jax import lax
from jax.experimental import pallas as pl
from jax.experimental.pallas import tpu as pltpu
from jax.experimental.pallas import tpu_sc as plsc

C = 1000


def kernel(x):
    B, S = x.shape
    N = B * S
    xf = x.reshape(N).astype(jnp.int32)

    info = plsc.get_sparse_core_info()
    NC, NS, L = info.num_cores, info.num_subcores, info.num_lanes
    NW = NC * NS
    RPW = N // NW
    G = 32
    NG = RPW // G
    NBUF = 2

    mesh = plsc.VectorSubcoreMesh(core_axis_name="c", subcore_axis_name="s")

    @functools.partial(
        pl.kernel,
        mesh=mesh,
        compiler_params=pltpu.CompilerParams(
            needs_layout_passes=False, use_tc_tiling_on_sc=True
        ),
        out_type=jax.ShapeDtypeStruct((N, C), jnp.float32),
        scratch_types=[
            pltpu.VMEM((RPW,), jnp.int32),
            pltpu.VMEM((NBUF * G, C), jnp.float32),
        ] + [pltpu.SemaphoreType.DMA] * NBUF,
    )
    def k(x_hbm, out_hbm, idx_v, buf_v, *sems):
        wid = lax.axis_index("s") * NC + lax.axis_index("c")
        base = wid * RPW
        pltpu.sync_copy(x_hbm.at[pl.ds(base, RPW)], idx_v)

        zeros = jnp.zeros((L,), jnp.float32)
        ones = jnp.ones((L,), jnp.float32)
        lane = lax.iota(jnp.int32, L)

        def zbody(i, carry):
            flat = i * L + lane
            plsc.store_scatter(buf_v, [flat // C, flat % C], zeros)
            return carry

        lax.fori_loop(0, (NBUF * G * C) // L, zbody, 0)

        def scatter_group(g, slot, val):
            for r in range(G // L):
                rows = lane + (slot * G + r * L)
                cols = idx_v[pl.ds(g * G + r * L, L)]
                plsc.store_scatter(buf_v, [rows, cols], val)

        def group_dma(g, slot):
            return pltpu.make_async_copy(
                buf_v.at[pl.ds(slot * G, G), :],
                out_hbm.at[pl.ds((base + g * G), G), :],
                sems[slot],
            )

        def gouter(i, carry):
            for b in range(NBUF):
                g = i * NBUF + b

                @pl.when(g >= NBUF)
                def _():
                    group_dma(g - NBUF, b).wait()
                    scatter_group(g - NBUF, b, zeros)

                scatter_group(g, b, ones)
                group_dma(g, b).start()
            return carry

        lax.fori_loop(0, NG // NBUF, gouter, 0)
        for b in range(NBUF):
            group_dma(NG - NBUF + b, b).wait()

    out = k(xf)
    return out.reshape(B, S, C)


# speedup vs baseline: 3.0130x; 3.0130x over previous
import functools

import jax
import jax.numpy as jnp
from jax import lax
from jax.experimental import pallas as pl
from jax.experimental.pallas import tpu as pltpu
from jax.experimental.pallas import tpu_sc as plsc

C = 1000
C0 = 504
BW = 128


def kernel(x):
    B, S = x.shape
    N = B * S
    xt = jnp.transpose(x).astype(jnp.int32).reshape(N)

    info = plsc.get_sparse_core_info()
    NC, NS, L = info.num_cores, info.num_subcores, info.num_lanes
    NW = NC * NS
    NGROUP = S * (B // BW)
    GPW = NGROUP // NW
    NBC = B // BW
    HOFF = (0, C0)
    HS = (C0, C - C0)

    mesh = plsc.VectorSubcoreMesh(core_axis_name="c", subcore_axis_name="s")

    @functools.partial(
        pl.kernel,
        mesh=mesh,
        compiler_params=pltpu.CompilerParams(
            needs_layout_passes=False, use_tc_tiling_on_sc=True
        ),
        out_type=jax.ShapeDtypeStruct((S * C, B), jnp.float32),
        scratch_types=[
            pltpu.VMEM((BW,), jnp.int32),
            pltpu.VMEM((BW,), jnp.int32),
            pltpu.VMEM((C0, BW), jnp.float32),
            pltpu.VMEM((C - C0, BW), jnp.float32),
            pltpu.SemaphoreType.DMA,
            pltpu.SemaphoreType.DMA,
            pltpu.SemaphoreType.DMA,
        ],
    )
    def k(xt_hbm, out_hbm, idx_a, idx_b, buf_a, buf_b, sem_a, sem_b, sem_idx):
        wid = lax.axis_index("s") * NC + lax.axis_index("c")
        idx_bufs = (idx_a, idx_b)
        bufs = (buf_a, buf_b)
        sems = (sem_a, sem_b)

        zeros = jnp.zeros((L,), jnp.float32)
        ones = jnp.ones((L,), jnp.float32)
        lane = lax.iota(jnp.int32, L)

        def zbody(c, carry):
            for h in range(2):
                @pl.when(c < HS[h])
                def _():
                    for r in range(BW // L):
                        bufs[h][c, pl.ds(r * L, L)] = zeros
            return carry

        lax.fori_loop(0, C0, zbody, 0)

        def load_ids(idx_v):
            return [idx_v[pl.ds(r * L, L)] for r in range(BW // L)]

        def scatter_half(ids, h, val):
            for r in range(BW // L):
                cols = lane + r * L
                rows = ids[r] - HOFF[h]
                mask = (rows >= 0) & (rows < HS[h])
                plsc.store_scatter(bufs[h], [rows, cols], val, mask=mask)

        def idx_dma(g, idx_v):
            return pltpu.make_async_copy(
                xt_hbm.at[pl.ds((wid * GPW + g) * BW, BW)], idx_v, sem_idx
            )

        def half_dma(g, h):
            gidx = wid * GPW + g
            s = gidx // NBC
            bc = gidx % NBC
            return pltpu.make_async_copy(
                bufs[h],
                out_hbm.at[pl.ds(s * C + HOFF[h], HS[h]),
                           pl.ds(bc * BW, BW)],
                sems[h],
            )

        idx_dma(0, idx_bufs[0]).start()
        idx_dma(0, idx_bufs[0]).wait()

        def gbody(i, carry):
            for p in range(2):
                g = i * 2 + p
                ids_old = load_ids(idx_bufs[1 - p])
                ids_cur = load_ids(idx_bufs[p])

                @pl.when(g + 1 < GPW)
                def _():
                    idx_dma(g + 1, idx_bufs[1 - p]).start()

                for h in range(2):
                    @pl.when(g >= 1)
                    def _():
                        half_dma(g - 1, h).wait()
                        scatter_half(ids_old, h, zeros)

                    scatter_half(ids_cur, h, ones)
                    half_dma(g, h).start()

                @pl.when(g + 1 < GPW)
                def _():
                    idx_dma(g + 1, idx_bufs[1 - p]).wait()

            return carry

        lax.fori_loop(0, GPW // 2, gbody, 0)
        for h in range(2):
            half_dma(GPW - 1, h).wait()

    out = k(xt)
    return jnp.transpose(out.reshape(S, C, B), (2, 0, 1))


# speedup vs baseline: 3.0334x; 1.0068x over previous
import functools

import jax
import jax.numpy as jnp
from jax import lax
from jax.experimental import pallas as pl
from jax.experimental.pallas import tpu as pltpu
from jax.experimental.pallas import tpu_sc as plsc

C = 1000
BW = 128


def kernel(x):
    B, S = x.shape
    N = B * S
    xt = jnp.transpose(x).astype(jnp.int32).reshape(N)

    info = plsc.get_sparse_core_info()
    NC, NS, L = info.num_cores, info.num_subcores, info.num_lanes
    NW = NC * NS
    NGROUP = S * (B // BW)
    GPW = NGROUP // NW
    NBC = B // BW

    mesh = plsc.VectorSubcoreMesh(core_axis_name="c", subcore_axis_name="s")

    @functools.partial(
        pl.kernel,
        mesh=mesh,
        compiler_params=pltpu.CompilerParams(
            needs_layout_passes=False, use_tc_tiling_on_sc=True
        ),
        out_type=jax.ShapeDtypeStruct((S * C, B), jnp.float32),
        scratch_types=[
            pltpu.VMEM((BW,), jnp.int32),
            pltpu.VMEM((BW,), jnp.int32),
            pltpu.VMEM((C, BW), jnp.float32),
            pltpu.SemaphoreType.DMA,
            pltpu.SemaphoreType.DMA,
        ],
    )
    def k(xt_hbm, out_hbm, idx_a, idx_b, buf_v, sem_big, sem_idx):
        wid = lax.axis_index("s") * NC + lax.axis_index("c")
        idx_bufs = (idx_a, idx_b)

        zeros = jnp.zeros((L,), jnp.float32)
        ones = jnp.ones((L,), jnp.float32)
        lane = lax.iota(jnp.int32, L)

        def zbody(c, carry):
            for r in range(BW // L):
                buf_v[c, pl.ds(r * L, L)] = zeros
            return carry

        lax.fori_loop(0, C, zbody, 0)

        def scatter_group(idx_v, val):
            for r in range(BW // L):
                cols = lane + r * L
                rows = idx_v[pl.ds(r * L, L)]
                plsc.store_scatter(buf_v, [rows, cols], val)

        def idx_dma(g, idx_v):
            return pltpu.make_async_copy(
                xt_hbm.at[pl.ds((wid * GPW + g) * BW, BW)], idx_v, sem_idx
            )

        idx_dma(0, idx_bufs[0]).start()
        idx_dma(0, idx_bufs[0]).wait()

        def gbody(i, carry):
            for p in range(2):
                g = i * 2 + p
                gidx = wid * GPW + g
                s = gidx // NBC
                bc = gidx % NBC
                cur, nxt = idx_bufs[p], idx_bufs[1 - p]

                @pl.when(g + 1 < GPW)
                def _():
                    idx_dma(g + 1, nxt).start()

                scatter_group(cur, ones)
                pltpu.async_copy(
                    buf_v,
                    out_hbm.at[pl.ds(s * C, C), pl.ds(bc * BW, BW)],
                    sem_big,
                ).wait()
                scatter_group(cur, zeros)

                @pl.when(g + 1 < GPW)
                def _():
                    idx_dma(g + 1, nxt).wait()

            return carry

        lax.fori_loop(0, GPW // 2, gbody, 0)

    out = k(xt)
    return jnp.transpose(out.reshape(S, C, B), (2, 0, 1))
